# Initial kernel scaffold; baseline (speedup 1.0000x reference)
import jax
import jax.numpy as jnp
from jax.experimental import pallas as pl


def kernel(x_congressperson, x_committee, edge_index, edge_attr, emb_cong, emb_comm, Wt_cong, bt_cong, Wt_comm, bt_comm, W1, b1, W2, b2, root, bias_conv, W_out, b_out):
    raise NotImplementedError("write your pallas kernel here")


# speedup vs baseline: 3.2791x; 3.2791x over previous
import functools

import jax
import jax.numpy as jnp
from jax import lax
from jax.experimental import pallas as pl
from jax.experimental.pallas import tpu as pltpu
from jax.experimental.pallas import tpu_sc as plsc

N_NODES = 10000
EMB = 16
OUT = 16
NEF = 4
HID = EMB * OUT
E = 160000

NC = 2
NS = 16
NW = NC * NS

G_PAD = 10240
G_PER_W = G_PAD // NW
G_CHUNKS, G_CW = 5, 64
E_PER_W = E // NW
E_CHUNKS, E_CW = 40, 125
EBLK = 1280
EGRID = E // EBLK
MSGW = 32

def _sc_mesh():
    return plsc.VectorSubcoreMesh(
        core_axis_name="c", subcore_axis_name="s",
        num_cores=NC, num_subcores=NS)


def _sc_params():
    return pltpu.CompilerParams(use_tc_tiling_on_sc=False)


def _worker_id():
    return lax.axis_index("s") * NC + lax.axis_index("c")


@functools.cache
def _emb_gather_fn():
    @functools.partial(
        pl.kernel,
        out_type=jax.ShapeDtypeStruct((G_PAD, EMB), jnp.float32),
        mesh=_sc_mesh(),
        compiler_params=_sc_params(),
        scratch_types=[
            pltpu.VMEM((G_CHUNKS, G_CW), jnp.int32),
            pltpu.VMEM((G_CW, EMB), jnp.float32),
            pltpu.SemaphoreType.DMA,
        ],
    )
    def _emb_gather(table_hbm, idx_hbm, out_hbm, idx_v, rows_v, sem):
        wid = _worker_id()
        pltpu.sync_copy(idx_hbm.at[wid], idx_v)
        for j in range(G_CHUNKS):
            pltpu.async_copy(table_hbm.at[idx_v.at[j]], rows_v, sem).wait()
            pltpu.sync_copy(
                rows_v, out_hbm.at[pl.ds(wid * G_PER_W + j * G_CW, G_CW)])

    return _emb_gather


def _xform_body(g_ref, wt_ref, bt_ref, x_ref):
    g = g_ref[0]
    y = jnp.dot(g[:N_NODES // 2], wt_ref[0],
                preferred_element_type=jnp.float32) + bt_ref[0]
    x_ref[...] = y


def _node_transform(g, wt, bt):
    half = N_NODES // 2
    return pl.pallas_call(
        _xform_body,
        grid=(2,),
        in_specs=[
            pl.BlockSpec((1, G_PAD // 2, EMB), lambda h: (h, 0, 0)),
            pl.BlockSpec((1, EMB, EMB), lambda h: (h, 0, 0)),
            pl.BlockSpec((1, 1, EMB), lambda h: (h, 0, 0)),
        ],
        out_specs=pl.BlockSpec((half, EMB), lambda h: (h, 0)),
        out_shape=jax.ShapeDtypeStruct((N_NODES, EMB), jnp.float32),
    )(g, wt, bt)


@functools.cache
def _src_gather_fn():
    @functools.partial(
        pl.kernel,
        out_type=jax.ShapeDtypeStruct((E, EMB), jnp.float32),
        mesh=_sc_mesh(),
        compiler_params=_sc_params(),
        scratch_types=[
            pltpu.VMEM((E_CHUNKS, E_CW), jnp.int32),
            pltpu.VMEM((E_CW, EMB), jnp.float32),
            pltpu.SemaphoreType.DMA,
        ],
    )
    def _src_gather(x_hbm, src_hbm, out_hbm, idx_v, rows_v, sem):
        wid = _worker_id()
        pltpu.sync_copy(src_hbm.at[wid], idx_v)

        def body(j, carry):
            pltpu.async_copy(x_hbm.at[idx_v.at[j]], rows_v, sem).wait()
            pltpu.sync_copy(
                rows_v, out_hbm.at[pl.ds(wid * E_PER_W + j * E_CW, E_CW)])
            return carry

        lax.fori_loop(0, E_CHUNKS, body, 0)

    return _src_gather


def _edge_body(ea_ref, xs_ref, w1_ref, b1_ref, w2_ref, b2_ref, out_ref):
    ea = ea_ref[...]
    h = jnp.dot(ea, w1_ref[...], preferred_element_type=jnp.float32)
    h = jnp.maximum(h + b1_ref[...], 0.0)
    w = jnp.dot(h, w2_ref[...], preferred_element_type=jnp.float32)
    w = w + b2_ref[...]
    xs = xs_ref[...]
    col = lax.broadcasted_iota(jnp.int32, (EMB, HID), 1)
    row = lax.broadcasted_iota(jnp.int32, (EMB, HID), 0)
    rmat = (col // OUT == row).astype(jnp.float32)
    colj = lax.broadcasted_iota(jnp.int32, (HID, OUT), 0)
    colo = lax.broadcasted_iota(jnp.int32, (HID, OUT), 1)
    smat = (colj % OUT == colo).astype(jnp.float32)
    xrep = jnp.dot(xs, rmat, preferred_element_type=jnp.float32)
    msg = jnp.dot(xrep * w, smat, preferred_element_type=jnp.float32)
    lane = lax.broadcasted_iota(jnp.int32, (EBLK, MSGW - OUT), 1)
    tail = (lane == 0).astype(jnp.float32)
    out_ref[...] = jnp.concatenate([msg, tail], axis=1)


def _edge_mlp(edge_attr, xsrc, w1, b1, w2, b2):
    return pl.pallas_call(
        _edge_body,
        grid=(EGRID,),
        in_specs=[
            pl.BlockSpec((EBLK, NEF), lambda i: (i, 0)),
            pl.BlockSpec((EBLK, EMB), lambda i: (i, 0)),
            pl.BlockSpec((NEF, HID), lambda i: (0, 0)),
            pl.BlockSpec((1, HID), lambda i: (0, 0)),
            pl.BlockSpec((HID, HID), lambda i: (0, 0)),
            pl.BlockSpec((1, HID), lambda i: (0, 0)),
        ],
        out_specs=pl.BlockSpec((EBLK, MSGW), lambda i: (i, 0)),
        out_shape=jax.ShapeDtypeStruct((E, MSGW), jnp.float32),
        compiler_params=pltpu.CompilerParams(
            dimension_semantics=("arbitrary",)),
    )(edge_attr, xsrc, w1, b1, w2, b2)


ROWS_PER_TILE = N_NODES // NS


@functools.cache
def _scatter_mean_fn():
    @functools.partial(
        pl.kernel,
        out_type=jax.ShapeDtypeStruct((NC, N_NODES, MSGW), jnp.float32),
        mesh=_sc_mesh(),
        compiler_params=_sc_params(),
        scratch_types=[
            pltpu.VMEM((E_CHUNKS, E_CW), jnp.int32),
            pltpu.VMEM((E_CW, MSGW), jnp.float32),
            pltpu.VMEM_SHARED((N_NODES, MSGW), jnp.float32),
        ],
    )
    def _scatter_mean(msg_hbm, dst_hbm, zeros_hbm, out_hbm, idx_v, buf_v, acc):
        cid = lax.axis_index("c")
        sid = lax.axis_index("s")
        wid = sid * NC + cid
        rbase = sid * ROWS_PER_TILE
        pltpu.sync_copy(zeros_hbm.at[pl.ds(rbase, ROWS_PER_TILE)],
                        acc.at[pl.ds(rbase, ROWS_PER_TILE)])
        pltpu.sync_copy(dst_hbm.at[wid], idx_v)
        plsc.subcore_barrier()

        def body(j, carry):
            pltpu.sync_copy(msg_hbm.at[wid, j], buf_v)
            pltpu.sync_copy(buf_v, acc.at[idx_v.at[j]], add=True)
            return carry

        lax.fori_loop(0, E_CHUNKS, body, 0)
        plsc.subcore_barrier()
        pltpu.sync_copy(acc.at[pl.ds(rbase, ROWS_PER_TILE)],
                        out_hbm.at[cid, pl.ds(rbase, ROWS_PER_TILE)])

    return _scatter_mean


def _final_body(p_ref, x_ref, root_ref, bc_ref, wo_ref, bo_ref, out_ref):
    s = p_ref[0] + p_ref[1]
    agg = s[:, :OUT]
    cnt = jnp.sum(s[:, OUT:], axis=1, keepdims=True)
    mean = agg / jnp.maximum(cnt, 1.0)
    xr = jnp.dot(x_ref[...], root_ref[...], preferred_element_type=jnp.float32)
    oc = mean + xr + bc_ref[...]
    x2 = jnp.maximum(oc, 0.0)
    out_ref[...] = jnp.dot(x2, wo_ref[...],
                           preferred_element_type=jnp.float32) + bo_ref[...]


def _finalize(partials, x, root, bias_conv, w_out, b_out):
    return pl.pallas_call(
        _final_body,
        out_shape=jax.ShapeDtypeStruct((N_NODES, 1), jnp.float32),
    )(partials, x, root.reshape(EMB, OUT), bias_conv.reshape(1, OUT),
      w_out.reshape(OUT, 1), b_out.reshape(1, 1))


def kernel(x_congressperson, x_committee, edge_index, edge_attr,
           emb_cong, emb_comm, Wt_cong, bt_cong, Wt_comm, bt_comm,
           W1, b1, W2, b2, root, bias_conv, W_out, b_out):
    half = N_NODES // 2
    pad = G_PAD // 2 - half
    n_cong = emb_cong.shape[0]

    table = jnp.concatenate([emb_cong, emb_comm], axis=0)
    zpad = jnp.zeros((pad,), jnp.int32)
    idx_all = jnp.concatenate(
        [x_congressperson, zpad, x_committee + n_cong, zpad]
    ).reshape(NW, G_CHUNKS, G_CW)
    g = _emb_gather_fn()(table, idx_all)

    wt = jnp.stack([Wt_cong, Wt_comm]).astype(jnp.float32)
    bt = jnp.stack([bt_cong, bt_comm]).astype(jnp.float32).reshape(2, 1, EMB)
    x = _node_transform(g.reshape(2, G_PAD // 2, EMB), wt, bt)

    src = edge_index[0].reshape(NW, E_CHUNKS, E_CW)
    xsrc = _src_gather_fn()(x, src)

    msg = _edge_mlp(edge_attr, xsrc, W1, b1.reshape(1, HID),
                    W2, b2.reshape(1, HID))

    dst = edge_index[1].reshape(NW, E_CHUNKS, E_CW)
    zeros = jnp.zeros((N_NODES, MSGW), jnp.float32)
    partials = _scatter_mean_fn()(
        msg.reshape(NW, E_CHUNKS, E_CW, MSGW), dst, zeros)

    return _finalize(partials, x, root, bias_conv, W_out, b_out)
